# baseline (speedup 1.0000x reference)
import functools

import jax
import jax.numpy as jnp
from jax import lax
from jax.experimental import pallas as pl
from jax.experimental.pallas import tpu as pltpu
from jax.experimental.pallas import tpu_sc as plsc

NW = 32
EP = 327680
CE = 1024
G = 128
NG = CE // G
BE = 8000


def _tables_body(vd_ref, ws_ref, wr_ref, ps_ref, pr_ref):
    vd = vd_ref[...]
    ps_ref[...] = jnp.dot(vd, ws_ref[...], preferred_element_type=jnp.float32)
    pr_ref[...] = jnp.dot(vd, wr_ref[...], preferred_element_type=jnp.float32)


def _sc_body(ps_hbm, pr_hbm, sids_hbm, rids_hbm, out_hbm,
             sidx_v, ridx_v, gs_v, gr_v, acc_v, sem_s, sem_r):
    ew = EP // NW
    nch = ew // CE
    wid = lax.axis_index("s") * 2 + lax.axis_index("c")

    for ch in range(nch):
        ebase = pl.multiple_of(wid * ew + ch * CE, 8)
        row0 = pl.multiple_of((wid * ew + ch * CE) // G, 8)
        pltpu.sync_copy(sids_hbm.at[pl.ds(row0, NG)], sidx_v)
        pltpu.sync_copy(rids_hbm.at[pl.ds(row0, NG)], ridx_v)
        cps = []
        for j in range(NG):
            cps.append(pltpu.async_copy(
                ps_hbm.at[sidx_v.at[j]], gs_v.at[pl.ds(j * G, G)], sem_s))
            cps.append(pltpu.async_copy(
                pr_hbm.at[ridx_v.at[j]], gr_v.at[pl.ds(j * G, G)], sem_r))
        for cp in cps:
            cp.wait()

        @plsc.parallel_loop(0, CE, 1, unroll=8)
        def _row(e):
            acc_v[e, :] = gs_v[e, :] + gr_v[e, :]

        pltpu.sync_copy(acc_v, out_hbm.at[pl.ds(ebase, CE)])


def _final_body(ed_ref, we_ref, b_ref, gsum_ref, out_ref):
    eproj = (
        jnp.dot(ed_ref[...], we_ref[...], preferred_element_type=jnp.float32)
        + b_ref[...]
    )
    out_ref[...] = jnp.maximum(eproj + gsum_ref[...], 0.0)


def kernel(vdata, edata, sender_ids, receiver_ids, W, b):
    Bn, N, DV = vdata.shape
    _, E, DE = edata.shape
    DOUT = W.shape[1]

    vd = vdata.reshape(N, DV)
    ed = edata.reshape(E, DE)
    pad = EP - E
    sid = jnp.pad(sender_ids.reshape(E), (0, pad)).reshape(EP // G, G)
    rid = jnp.pad(receiver_ids.reshape(E), (0, pad)).reshape(EP // G, G)
    we = W[:DE]
    ws = W[DE:DE + DV]
    wr = W[DE + DV:]

    ps, pr = pl.pallas_call(
        _tables_body,
        out_shape=[jax.ShapeDtypeStruct((N, DOUT), jnp.float32)] * 2,
    )(vd, ws, wr)

    mesh = plsc.VectorSubcoreMesh(core_axis_name="c", subcore_axis_name="s")
    sc_call = pl.kernel(
        _sc_body,
        out_type=jax.ShapeDtypeStruct((EP, 16), jnp.float32),
        mesh=mesh,
        compiler_params=pltpu.CompilerParams(use_tc_tiling_on_sc=False),
        scratch_types=[
            pltpu.VMEM((NG, G), jnp.int32),
            pltpu.VMEM((NG, G), jnp.int32),
            pltpu.VMEM((CE, 16), jnp.float32),
            pltpu.VMEM((CE, 16), jnp.float32),
            pltpu.VMEM((CE, 16), jnp.float32),
            pltpu.SemaphoreType.DMA,
            pltpu.SemaphoreType.DMA,
        ],
    )
    gsum = sc_call(ps, pr, sid, rid)

    out = pl.pallas_call(
        _final_body,
        grid=(E // BE,),
        in_specs=[
            pl.BlockSpec((BE, DE), lambda i: (i, 0)),
            pl.BlockSpec((DE, DOUT), lambda i: (0, 0)),
            pl.BlockSpec((1, DOUT), lambda i: (0, 0)),
            pl.BlockSpec((BE, 16), lambda i: (i, 0)),
        ],
        out_specs=pl.BlockSpec((BE, DOUT), lambda i: (i, 0)),
        out_shape=jax.ShapeDtypeStruct((E, DOUT), jnp.float32),
    )(ed, we, b.reshape(1, DOUT), gsum)
    return out.reshape(Bn, E, DOUT)


# speedup vs baseline: 1.1848x; 1.1848x over previous
import functools

import jax
import jax.numpy as jnp
from jax import lax
from jax.experimental import pallas as pl
from jax.experimental.pallas import tpu as pltpu
from jax.experimental.pallas import tpu_sc as plsc

NW = 32
EP = 327680
CE = 1024
CP = CE // 8
G = 128
NG = CE // G
BEF = 1000


def _tables_body(vd_ref, ws_ref, wr_ref, ps_ref, pr_ref):
    vd = vd_ref[...]
    ps_ref[...] = jnp.dot(vd, ws_ref[...], preferred_element_type=jnp.float32)
    pr_ref[...] = jnp.dot(vd, wr_ref[...], preferred_element_type=jnp.float32)


def _sc_body(ps_hbm, pr_hbm, sids_hbm, rids_hbm, out_hbm,
             sidx_v, ridx_v, gs_v, gr_v, acc_v,
             sem_s0, sem_s1, sem_r0, sem_r1, sem_o0, sem_o1):
    ew = EP // NW
    nch = ew // CE
    wid = lax.axis_index("s") * 2 + lax.axis_index("c")
    sem_s = (sem_s0, sem_s1)
    sem_r = (sem_r0, sem_r1)
    sem_o = (sem_o0, sem_o1)

    def fire_gathers(ch):
        p = ch % 2
        row0 = pl.multiple_of((wid * ew + ch * CE) // G, 8)
        pltpu.sync_copy(sids_hbm.at[pl.ds(row0, NG)], sidx_v.at[p])
        pltpu.sync_copy(rids_hbm.at[pl.ds(row0, NG)], ridx_v.at[p])
        cps = []
        for j in range(NG):
            cps.append(pltpu.async_copy(
                ps_hbm.at[sidx_v.at[p, j]],
                gs_v.at[p, pl.ds(j * G, G)], sem_s[p]))
            cps.append(pltpu.async_copy(
                pr_hbm.at[ridx_v.at[p, j]],
                gr_v.at[p, pl.ds(j * G, G)], sem_r[p]))
        return cps

    gath = fire_gathers(0)
    outw = {}
    for ch in range(nch):
        p = ch % 2
        nxt = fire_gathers(ch + 1) if ch + 1 < nch else []
        for cp in gath:
            cp.wait()
        gath = nxt
        if ch >= 2:
            outw[ch - 2].wait()

        @plsc.parallel_loop(0, CP, 1, unroll=2)
        def _row(g):
            for k in range(8):
                e = g * 8 + k
                acc_v[p, g, pl.ds(16 * k, 16)] = (
                    gs_v[p, e, :] + gr_v[p, e, :])

        prow0 = pl.multiple_of((wid * ew + ch * CE) // 8, 8)
        outw[ch] = pltpu.async_copy(
            acc_v.at[p], out_hbm.at[pl.ds(prow0, CP)], sem_o[p])
    outw[nch - 2].wait()
    outw[nch - 1].wait()


def _final_body(edp_ref, wblk_ref, bp_ref, gsum_ref, out_ref):
    eproj = (
        jnp.dot(edp_ref[...], wblk_ref[...],
                preferred_element_type=jnp.float32)
        + bp_ref[...]
    )
    out_ref[...] = jnp.maximum(eproj + gsum_ref[...], 0.0)


def kernel(vdata, edata, sender_ids, receiver_ids, W, b):
    Bn, N, DV = vdata.shape
    _, E, DE = edata.shape
    DOUT = W.shape[1]

    vd = vdata.reshape(N, DV)
    edp = edata.reshape(E * DE // 128, 128)
    pad = EP - E
    sid = jnp.pad(sender_ids.reshape(E), (0, pad)).reshape(EP // G, G)
    rid = jnp.pad(receiver_ids.reshape(E), (0, pad)).reshape(EP // G, G)
    we = W[:DE]
    ws = W[DE:DE + DV]
    wr = W[DE + DV:]
    wblk = jax.scipy.linalg.block_diag(*([we] * 8))
    bp = jnp.tile(b, 8).reshape(1, 128)

    ps, pr = pl.pallas_call(
        _tables_body,
        out_shape=[jax.ShapeDtypeStruct((N, DOUT), jnp.float32)] * 2,
    )(vd, ws, wr)

    mesh = plsc.VectorSubcoreMesh(core_axis_name="c", subcore_axis_name="s")
    sc_call = pl.kernel(
        _sc_body,
        out_type=jax.ShapeDtypeStruct((EP // 8, 128), jnp.float32),
        mesh=mesh,
        compiler_params=pltpu.CompilerParams(use_tc_tiling_on_sc=False),
        scratch_types=[
            pltpu.VMEM((2, NG, G), jnp.int32),
            pltpu.VMEM((2, NG, G), jnp.int32),
            pltpu.VMEM((2, CE, 16), jnp.float32),
            pltpu.VMEM((2, CE, 16), jnp.float32),
            pltpu.VMEM((2, CP, 128), jnp.float32),
            pltpu.SemaphoreType.DMA,
            pltpu.SemaphoreType.DMA,
            pltpu.SemaphoreType.DMA,
            pltpu.SemaphoreType.DMA,
            pltpu.SemaphoreType.DMA,
            pltpu.SemaphoreType.DMA,
        ],
    )
    gsum = sc_call(ps, pr, sid, rid)

    npk = E * DE // 128
    outp = pl.pallas_call(
        _final_body,
        grid=(npk // BEF,),
        in_specs=[
            pl.BlockSpec((BEF, 128), lambda i: (i, 0)),
            pl.BlockSpec((128, 128), lambda i: (0, 0)),
            pl.BlockSpec((1, 128), lambda i: (0, 0)),
            pl.BlockSpec((BEF, 128), lambda i: (i, 0)),
        ],
        out_specs=pl.BlockSpec((BEF, 128), lambda i: (i, 0)),
        out_shape=jax.ShapeDtypeStruct((npk, 128), jnp.float32),
    )(edp, wblk, bp, gsum)
    return outp.reshape(Bn, E, DOUT)


# speedup vs baseline: 2.1635x; 1.8260x over previous
import functools

import jax
import jax.numpy as jnp
from jax import lax
from jax.experimental import pallas as pl
from jax.experimental.pallas import tpu as pltpu
from jax.experimental.pallas import tpu_sc as plsc

NW = 32
EP = 327680
CE = 1024
CP = CE // 8
G = 128
NG = CE // G
BEF = 1024


def _tables_body(vd_ref, ws_ref, wr_ref, b_ref, ps_ref, pr_ref):
    vd = vd_ref[...]
    ps_ref[...] = (
        jnp.dot(vd, ws_ref[...], preferred_element_type=jnp.float32)
        + b_ref[...])
    pr_ref[...] = jnp.dot(vd, wr_ref[...], preferred_element_type=jnp.float32)


def _sc_body(ps_hbm, pr_hbm, sids_hbm, rids_hbm, out_hbm,
             sidx_v, ridx_v, gs_v, gr_v, acc_v,
             sem_s0, sem_s1, sem_r0, sem_r1, sem_o0, sem_o1):
    ew = EP // NW
    nch = ew // CE
    wid = lax.axis_index("s") * 2 + lax.axis_index("c")
    sem_s = (sem_s0, sem_s1)
    sem_r = (sem_r0, sem_r1)
    sem_o = (sem_o0, sem_o1)

    def fire_gathers(ch):
        p = ch % 2
        row0 = pl.multiple_of((wid * ew + ch * CE) // G, 8)
        pltpu.sync_copy(sids_hbm.at[pl.ds(row0, NG)], sidx_v.at[p])
        pltpu.sync_copy(rids_hbm.at[pl.ds(row0, NG)], ridx_v.at[p])
        cps = []
        for j in range(NG):
            cps.append(pltpu.async_copy(
                ps_hbm.at[sidx_v.at[p, j]],
                gs_v.at[p, pl.ds(j * G, G)], sem_s[p]))
            cps.append(pltpu.async_copy(
                pr_hbm.at[ridx_v.at[p, j]],
                gr_v.at[p, pl.ds(j * G, G)], sem_r[p]))
        return cps

    gath = fire_gathers(0)
    outw = {}
    for ch in range(nch):
        p = ch % 2
        nxt = fire_gathers(ch + 1) if ch + 1 < nch else []
        for cp in gath:
            cp.wait()
        gath = nxt
        if ch >= 2:
            outw[ch - 2].wait()

        @plsc.parallel_loop(0, CP, 1, unroll=2)
        def _row(g):
            for k in range(8):
                e = g * 8 + k
                acc_v[p, g, pl.ds(16 * k, 16)] = (
                    gs_v[p, e, :] + gr_v[p, e, :])

        prow0 = pl.multiple_of((wid * ew + ch * CE) // 8, 8)
        outw[ch] = pltpu.async_copy(
            acc_v.at[p], out_hbm.at[pl.ds(prow0, CP)], sem_o[p])
    outw[nch - 2].wait()
    outw[nch - 1].wait()


def _final_body(edt_ref, wet_ref, gsum_ref, out_ref):
    eproj = jnp.dot(wet_ref[...], edt_ref[...],
                    preferred_element_type=jnp.float32)
    gpt = gsum_ref[...].T
    gt = jnp.concatenate(
        [gpt[16 * k:16 * (k + 1), :] for k in range(8)], axis=1)
    out_ref[...] = jnp.maximum(eproj + gt, 0.0)


def kernel(vdata, edata, sender_ids, receiver_ids, W, b):
    Bn, N, DV = vdata.shape
    _, E, DE = edata.shape
    DOUT = W.shape[1]

    vd = vdata.reshape(N, DV)
    edt = edata.reshape(E, DE).T
    pad = EP - E
    be = BEF * 8

    def permute_ids(ids):
        p = jnp.pad(ids.reshape(E), (0, pad))
        return p.reshape(EP // be, 8, BEF).transpose(0, 2, 1).reshape(
            EP // G, G)

    sid = permute_ids(sender_ids)
    rid = permute_ids(receiver_ids)
    we = W[:DE]
    ws = W[DE:DE + DV]
    wr = W[DE + DV:]
    wet = we.T
    brow = b.reshape(1, DOUT)

    ps, pr = pl.pallas_call(
        _tables_body,
        out_shape=[jax.ShapeDtypeStruct((N, DOUT), jnp.float32)] * 2,
    )(vd, ws, wr, brow)

    mesh = plsc.VectorSubcoreMesh(core_axis_name="c", subcore_axis_name="s")
    sc_call = pl.kernel(
        _sc_body,
        out_type=jax.ShapeDtypeStruct((EP // 8, 128), jnp.float32),
        mesh=mesh,
        compiler_params=pltpu.CompilerParams(use_tc_tiling_on_sc=False),
        scratch_types=[
            pltpu.VMEM((2, NG, G), jnp.int32),
            pltpu.VMEM((2, NG, G), jnp.int32),
            pltpu.VMEM((2, CE, 16), jnp.float32),
            pltpu.VMEM((2, CE, 16), jnp.float32),
            pltpu.VMEM((2, CP, 128), jnp.float32),
            pltpu.SemaphoreType.DMA,
            pltpu.SemaphoreType.DMA,
            pltpu.SemaphoreType.DMA,
            pltpu.SemaphoreType.DMA,
            pltpu.SemaphoreType.DMA,
            pltpu.SemaphoreType.DMA,
        ],
    )
    gsum = sc_call(ps, pr, sid, rid)

    outt = pl.pallas_call(
        _final_body,
        grid=(pl.cdiv(E, be),),
        in_specs=[
            pl.BlockSpec((DE, be), lambda i: (0, i)),
            pl.BlockSpec((DOUT, DE), lambda i: (0, 0)),
            pl.BlockSpec((BEF, 128), lambda i: (i, 0)),
        ],
        out_specs=pl.BlockSpec((DOUT, be), lambda i: (0, i)),
        out_shape=jax.ShapeDtypeStruct((DOUT, E), jnp.float32),
    )(edt, wet, gsum)
    return outt.T.reshape(Bn, E, DOUT)


# speedup vs baseline: 3.4401x; 1.5901x over previous
import functools

import jax
import jax.numpy as jnp
from jax import lax
from jax.experimental import pallas as pl
from jax.experimental.pallas import tpu as pltpu
from jax.experimental.pallas import tpu_sc as plsc

NW = 32
EP = 327680
CE = 1024
CP = CE // 8
G = 128
NG = CE // G
BEF = 1024
E_EDGES = 320000


def _tables_body(vd_ref, ws_ref, wr_ref, b_ref, ps_ref, pr_ref):
    vd = vd_ref[...]
    ps_ref[...] = (
        jnp.dot(vd, ws_ref[...], preferred_element_type=jnp.float32)
        + b_ref[...])
    pr_ref[...] = jnp.dot(vd, wr_ref[...], preferred_element_type=jnp.float32)


def _sc_body(ps_hbm, pr_hbm, sids_hbm, rids_hbm, out_hbm,
             sidx_v, ridx_v, gs_v, gr_v, acc_v,
             sem_s0, sem_s1, sem_r0, sem_r1, sem_o0, sem_o1,
             sem_i0, sem_i1):
    ew = EP // NW
    nch = ew // CE
    wid = lax.axis_index("s") * 2 + lax.axis_index("c")
    sem_s = (sem_s0, sem_s1)
    sem_r = (sem_r0, sem_r1)
    sem_o = (sem_o0, sem_o1)
    sem_i = (sem_i0, sem_i1)

    def fire_gathers(ch):
        p = ch % 2
        r0 = wid * (ew // 8) + ch * CP
        blk = r0 // BEF
        rloc = r0 % BEF
        cps = []
        for k in range(NG):
            start = blk * (BEF * 8) + k * BEF + rloc
            start = jnp.where(start >= E_EDGES, 0, start)
            start = pl.multiple_of(start, 8)
            cps.append(pltpu.async_copy(
                sids_hbm.at[pl.ds(start, G)], sidx_v.at[p, k], sem_i[p]))
            cps.append(pltpu.async_copy(
                rids_hbm.at[pl.ds(start, G)], ridx_v.at[p, k], sem_i[p]))
        for cp in cps:
            cp.wait()
        cps = []
        for j in range(NG):
            cps.append(pltpu.async_copy(
                ps_hbm.at[sidx_v.at[p, j]],
                gs_v.at[p, pl.ds(j * G, G)], sem_s[p]))
            cps.append(pltpu.async_copy(
                pr_hbm.at[ridx_v.at[p, j]],
                gr_v.at[p, pl.ds(j * G, G)], sem_r[p]))
        return cps

    gath = fire_gathers(0)
    outw = {}
    for ch in range(nch):
        p = ch % 2
        nxt = fire_gathers(ch + 1) if ch + 1 < nch else []
        for cp in gath:
            cp.wait()
        gath = nxt
        if ch >= 2:
            outw[ch - 2].wait()

        @plsc.parallel_loop(0, CP, 1, unroll=2)
        def _row(g):
            for k in range(8):
                e = k * G + g
                acc_v[p, g, pl.ds(16 * k, 16)] = (
                    gs_v[p, e, :] + gr_v[p, e, :])

        prow0 = pl.multiple_of((wid * ew + ch * CE) // 8, 8)
        outw[ch] = pltpu.async_copy(
            acc_v.at[p], out_hbm.at[pl.ds(prow0, CP)], sem_o[p])
    outw[nch - 2].wait()
    outw[nch - 1].wait()


def _final_body(edt_ref, wet_ref, gsum_ref, out_ref):
    eproj = jnp.dot(wet_ref[...], edt_ref[...],
                    preferred_element_type=jnp.float32)
    gpt = gsum_ref[...].T
    gt = jnp.concatenate(
        [gpt[16 * k:16 * (k + 1), :] for k in range(8)], axis=1)
    out_ref[...] = jnp.maximum(eproj + gt, 0.0)


def kernel(vdata, edata, sender_ids, receiver_ids, W, b):
    Bn, N, DV = vdata.shape
    _, E, DE = edata.shape
    DOUT = W.shape[1]

    vd = vdata.reshape(N, DV)
    edt = edata.reshape(E, DE).T
    be = BEF * 8
    sid = sender_ids.reshape(E)
    rid = receiver_ids.reshape(E)
    we = W[:DE]
    ws = W[DE:DE + DV]
    wr = W[DE + DV:]
    wet = we.T
    brow = b.reshape(1, DOUT)

    ps, pr = pl.pallas_call(
        _tables_body,
        out_shape=[jax.ShapeDtypeStruct((N, DOUT), jnp.float32)] * 2,
    )(vd, ws, wr, brow)

    mesh = plsc.VectorSubcoreMesh(core_axis_name="c", subcore_axis_name="s")
    sc_call = pl.kernel(
        _sc_body,
        out_type=jax.ShapeDtypeStruct((EP // 8, 128), jnp.float32),
        mesh=mesh,
        compiler_params=pltpu.CompilerParams(use_tc_tiling_on_sc=False),
        scratch_types=[
            pltpu.VMEM((2, NG, G), jnp.int32),
            pltpu.VMEM((2, NG, G), jnp.int32),
            pltpu.VMEM((2, CE, 16), jnp.float32),
            pltpu.VMEM((2, CE, 16), jnp.float32),
            pltpu.VMEM((2, CP, 128), jnp.float32),
            pltpu.SemaphoreType.DMA,
            pltpu.SemaphoreType.DMA,
            pltpu.SemaphoreType.DMA,
            pltpu.SemaphoreType.DMA,
            pltpu.SemaphoreType.DMA,
            pltpu.SemaphoreType.DMA,
            pltpu.SemaphoreType.DMA,
            pltpu.SemaphoreType.DMA,
        ],
    )
    gsum = sc_call(ps, pr, sid, rid)

    outt = pl.pallas_call(
        _final_body,
        grid=(pl.cdiv(E, be),),
        in_specs=[
            pl.BlockSpec((DE, be), lambda i: (0, i)),
            pl.BlockSpec((DOUT, DE), lambda i: (0, 0)),
            pl.BlockSpec((BEF, 128), lambda i: (i, 0)),
        ],
        out_specs=pl.BlockSpec((DOUT, be), lambda i: (0, i)),
        out_shape=jax.ShapeDtypeStruct((DOUT, E), jnp.float32),
    )(edt, wet, gsum)
    return outt.T.reshape(Bn, E, DOUT)


# speedup vs baseline: 3.7280x; 1.0837x over previous
import functools

import jax
import jax.numpy as jnp
from jax import lax
from jax.experimental import pallas as pl
from jax.experimental.pallas import tpu as pltpu
from jax.experimental.pallas import tpu_sc as plsc

NW = 32
EP = 327680
CE = 1024
CP = CE // 8
G = 128
NG = CE // G
BEF = 2048
E_EDGES = 320000


def _tables_body(vd_ref, ws_ref, wr_ref, b_ref, ps_ref, pr_ref):
    vd = vd_ref[...]
    ps_ref[...] = (
        jnp.dot(vd, ws_ref[...], preferred_element_type=jnp.float32)
        + b_ref[...])
    pr_ref[...] = jnp.dot(vd, wr_ref[...], preferred_element_type=jnp.float32)


def _sc_body(ps_hbm, pr_hbm, sids_hbm, rids_hbm, out_hbm,
             sidx_v, ridx_v, gs_v, gr_v, acc_v,
             sem_s0, sem_s1, sem_r0, sem_r1, sem_o0, sem_o1,
             sem_i0, sem_i1):
    ew = EP // NW
    nch = ew // CE
    wid = lax.axis_index("s") * 2 + lax.axis_index("c")
    sem_s = (sem_s0, sem_s1)
    sem_r = (sem_r0, sem_r1)
    sem_o = (sem_o0, sem_o1)
    sem_i = (sem_i0, sem_i1)

    def fire_gathers(ch):
        p = ch % 2
        r0 = wid * (ew // 8) + ch * CP
        blk = r0 // BEF
        rloc = r0 % BEF
        cps = []
        for k in range(NG):
            start = blk * (BEF * 8) + k * BEF + rloc
            start = jnp.where(start >= E_EDGES, 0, start)
            start = pl.multiple_of(start, 8)
            cps.append(pltpu.async_copy(
                sids_hbm.at[pl.ds(start, G)], sidx_v.at[p, k], sem_i[p]))
            cps.append(pltpu.async_copy(
                rids_hbm.at[pl.ds(start, G)], ridx_v.at[p, k], sem_i[p]))
        for cp in cps:
            cp.wait()
        cps = []
        for j in range(NG):
            cps.append(pltpu.async_copy(
                ps_hbm.at[sidx_v.at[p, j]],
                gs_v.at[p, pl.ds(j * G, G)], sem_s[p]))
            cps.append(pltpu.async_copy(
                pr_hbm.at[ridx_v.at[p, j]],
                gr_v.at[p, pl.ds(j * G, G)], sem_r[p]))
        return cps

    gath = fire_gathers(0)
    outw = {}
    for ch in range(nch):
        p = ch % 2
        nxt = fire_gathers(ch + 1) if ch + 1 < nch else []
        for cp in gath:
            cp.wait()
        gath = nxt
        if ch >= 2:
            outw[ch - 2].wait()

        @plsc.parallel_loop(0, CP, 1, unroll=2)
        def _row(g):
            for k in range(8):
                e = k * G + g
                acc_v[p, g, pl.ds(16 * k, 16)] = (
                    gs_v[p, e, :] + gr_v[p, e, :])

        prow0 = pl.multiple_of((wid * ew + ch * CE) // 8, 8)
        outw[ch] = pltpu.async_copy(
            acc_v.at[p], out_hbm.at[pl.ds(prow0, CP)], sem_o[p])
    outw[nch - 2].wait()
    outw[nch - 1].wait()


def _final_body(edt_ref, wet_ref, gsum_ref, out_ref):
    eproj = jnp.dot(wet_ref[...], edt_ref[...],
                    preferred_element_type=jnp.float32)
    gpt = gsum_ref[...].T
    gt = jnp.concatenate(
        [gpt[16 * k:16 * (k + 1), :] for k in range(8)], axis=1)
    out_ref[...] = jnp.maximum(eproj + gt, 0.0)


def kernel(vdata, edata, sender_ids, receiver_ids, W, b):
    Bn, N, DV = vdata.shape
    _, E, DE = edata.shape
    DOUT = W.shape[1]

    vd = vdata.reshape(N, DV)
    edt = edata.reshape(E, DE).T
    be = BEF * 8
    sid = sender_ids.reshape(E)
    rid = receiver_ids.reshape(E)
    we = W[:DE]
    ws = W[DE:DE + DV]
    wr = W[DE + DV:]
    wet = we.T
    brow = b.reshape(1, DOUT)

    ps, pr = pl.pallas_call(
        _tables_body,
        out_shape=[jax.ShapeDtypeStruct((N, DOUT), jnp.float32)] * 2,
    )(vd, ws, wr, brow)

    mesh = plsc.VectorSubcoreMesh(core_axis_name="c", subcore_axis_name="s")
    sc_call = pl.kernel(
        _sc_body,
        out_type=jax.ShapeDtypeStruct((EP // 8, 128), jnp.float32),
        mesh=mesh,
        compiler_params=pltpu.CompilerParams(use_tc_tiling_on_sc=False),
        scratch_types=[
            pltpu.VMEM((2, NG, G), jnp.int32),
            pltpu.VMEM((2, NG, G), jnp.int32),
            pltpu.VMEM((2, CE, 16), jnp.float32),
            pltpu.VMEM((2, CE, 16), jnp.float32),
            pltpu.VMEM((2, CP, 128), jnp.float32),
            pltpu.SemaphoreType.DMA,
            pltpu.SemaphoreType.DMA,
            pltpu.SemaphoreType.DMA,
            pltpu.SemaphoreType.DMA,
            pltpu.SemaphoreType.DMA,
            pltpu.SemaphoreType.DMA,
            pltpu.SemaphoreType.DMA,
            pltpu.SemaphoreType.DMA,
        ],
    )
    gsum = sc_call(ps, pr, sid, rid)

    outt = pl.pallas_call(
        _final_body,
        grid=(pl.cdiv(E, be),),
        in_specs=[
            pl.BlockSpec((DE, be), lambda i: (0, i)),
            pl.BlockSpec((DOUT, DE), lambda i: (0, 0)),
            pl.BlockSpec((BEF, 128), lambda i: (i, 0)),
        ],
        out_specs=pl.BlockSpec((DOUT, be), lambda i: (0, i)),
        out_shape=jax.ShapeDtypeStruct((DOUT, E), jnp.float32),
    )(edt, wet, gsum)
    return outt.T.reshape(Bn, E, DOUT)


# speedup vs baseline: 3.8618x; 1.0359x over previous
import functools

import jax
import jax.numpy as jnp
from jax import lax
from jax.experimental import pallas as pl
from jax.experimental.pallas import tpu as pltpu
from jax.experimental.pallas import tpu_sc as plsc

NW = 32
EP = 327680
CE = 1024
CP = CE // 8
G = 128
NG = CE // G
BEF = 4096
E_EDGES = 320000


def _tables_body(vd_ref, ws_ref, wr_ref, b_ref, ps_ref, pr_ref):
    vd = vd_ref[...]
    ps_ref[...] = (
        jnp.dot(vd, ws_ref[...], preferred_element_type=jnp.float32)
        + b_ref[...])
    pr_ref[...] = jnp.dot(vd, wr_ref[...], preferred_element_type=jnp.float32)


def _sc_body(ps_hbm, pr_hbm, sids_hbm, rids_hbm, out_hbm,
             sidx_v, ridx_v, gs_v, gr_v, acc_v,
             sem_s0, sem_s1, sem_r0, sem_r1, sem_o0, sem_o1,
             sem_i0, sem_i1):
    ew = EP // NW
    nch = ew // CE
    wid = lax.axis_index("s") * 2 + lax.axis_index("c")
    sem_s = (sem_s0, sem_s1)
    sem_r = (sem_r0, sem_r1)
    sem_o = (sem_o0, sem_o1)
    sem_i = (sem_i0, sem_i1)

    def fire_gathers(ch):
        p = ch % 2
        r0 = wid * (ew // 8) + ch * CP
        blk = r0 // BEF
        rloc = r0 % BEF
        cps = []
        for k in range(NG):
            start = blk * (BEF * 8) + k * BEF + rloc
            start = jnp.where(start >= E_EDGES, 0, start)
            start = pl.multiple_of(start, 8)
            cps.append(pltpu.async_copy(
                sids_hbm.at[pl.ds(start, G)], sidx_v.at[p, k], sem_i[p]))
            cps.append(pltpu.async_copy(
                rids_hbm.at[pl.ds(start, G)], ridx_v.at[p, k], sem_i[p]))
        for cp in cps:
            cp.wait()
        cps = []
        for j in range(NG):
            cps.append(pltpu.async_copy(
                ps_hbm.at[sidx_v.at[p, j]],
                gs_v.at[p, pl.ds(j * G, G)], sem_s[p]))
            cps.append(pltpu.async_copy(
                pr_hbm.at[ridx_v.at[p, j]],
                gr_v.at[p, pl.ds(j * G, G)], sem_r[p]))
        return cps

    gath = fire_gathers(0)
    outw = {}
    for ch in range(nch):
        p = ch % 2
        nxt = fire_gathers(ch + 1) if ch + 1 < nch else []
        for cp in gath:
            cp.wait()
        gath = nxt
        if ch >= 2:
            outw[ch - 2].wait()

        @plsc.parallel_loop(0, CP, 1, unroll=2)
        def _row(g):
            for k in range(8):
                e = k * G + g
                acc_v[p, g, pl.ds(16 * k, 16)] = (
                    gs_v[p, e, :] + gr_v[p, e, :])

        prow0 = pl.multiple_of((wid * ew + ch * CE) // 8, 8)
        outw[ch] = pltpu.async_copy(
            acc_v.at[p], out_hbm.at[pl.ds(prow0, CP)], sem_o[p])
    outw[nch - 2].wait()
    outw[nch - 1].wait()


def _final_body(edt_ref, wet_ref, gsum_ref, out_ref):
    eproj = jnp.dot(wet_ref[...], edt_ref[...],
                    preferred_element_type=jnp.float32)
    gpt = gsum_ref[...].T
    gt = jnp.concatenate(
        [gpt[16 * k:16 * (k + 1), :] for k in range(8)], axis=1)
    out_ref[...] = jnp.maximum(eproj + gt, 0.0)


def kernel(vdata, edata, sender_ids, receiver_ids, W, b):
    Bn, N, DV = vdata.shape
    _, E, DE = edata.shape
    DOUT = W.shape[1]

    vd = vdata.reshape(N, DV)
    edt = edata.reshape(E, DE).T
    be = BEF * 8
    sid = sender_ids.reshape(E)
    rid = receiver_ids.reshape(E)
    we = W[:DE]
    ws = W[DE:DE + DV]
    wr = W[DE + DV:]
    wet = we.T
    brow = b.reshape(1, DOUT)

    ps, pr = pl.pallas_call(
        _tables_body,
        out_shape=[jax.ShapeDtypeStruct((N, DOUT), jnp.float32)] * 2,
    )(vd, ws, wr, brow)

    mesh = plsc.VectorSubcoreMesh(core_axis_name="c", subcore_axis_name="s")
    sc_call = pl.kernel(
        _sc_body,
        out_type=jax.ShapeDtypeStruct((EP // 8, 128), jnp.float32),
        mesh=mesh,
        compiler_params=pltpu.CompilerParams(use_tc_tiling_on_sc=False),
        scratch_types=[
            pltpu.VMEM((2, NG, G), jnp.int32),
            pltpu.VMEM((2, NG, G), jnp.int32),
            pltpu.VMEM((2, CE, 16), jnp.float32),
            pltpu.VMEM((2, CE, 16), jnp.float32),
            pltpu.VMEM((2, CP, 128), jnp.float32),
            pltpu.SemaphoreType.DMA,
            pltpu.SemaphoreType.DMA,
            pltpu.SemaphoreType.DMA,
            pltpu.SemaphoreType.DMA,
            pltpu.SemaphoreType.DMA,
            pltpu.SemaphoreType.DMA,
            pltpu.SemaphoreType.DMA,
            pltpu.SemaphoreType.DMA,
        ],
    )
    gsum = sc_call(ps, pr, sid, rid)

    outt = pl.pallas_call(
        _final_body,
        grid=(pl.cdiv(E, be),),
        in_specs=[
            pl.BlockSpec((DE, be), lambda i: (0, i)),
            pl.BlockSpec((DOUT, DE), lambda i: (0, 0)),
            pl.BlockSpec((BEF, 128), lambda i: (i, 0)),
        ],
        out_specs=pl.BlockSpec((DOUT, be), lambda i: (0, i)),
        out_shape=jax.ShapeDtypeStruct((DOUT, E), jnp.float32),
    )(edt, wet, gsum)
    return outt.T.reshape(Bn, E, DOUT)


# speedup vs baseline: 3.8629x; 1.0003x over previous
import functools

import jax
import jax.numpy as jnp
from jax import lax
from jax.experimental import pallas as pl
from jax.experimental.pallas import tpu as pltpu
from jax.experimental.pallas import tpu_sc as plsc

NW = 32
EP = 327680
CE = 1024
CP = CE // 8
G = 128
NG = CE // G
BEF = 8192
E_EDGES = 320000


def _tables_body(vd_ref, ws_ref, wr_ref, b_ref, ps_ref, pr_ref):
    vd = vd_ref[...]
    ps_ref[...] = (
        jnp.dot(vd, ws_ref[...], preferred_element_type=jnp.float32)
        + b_ref[...])
    pr_ref[...] = jnp.dot(vd, wr_ref[...], preferred_element_type=jnp.float32)


def _sc_body(ps_hbm, pr_hbm, sids_hbm, rids_hbm, out_hbm,
             sidx_v, ridx_v, gs_v, gr_v, acc_v,
             sem_s0, sem_s1, sem_r0, sem_r1, sem_o0, sem_o1,
             sem_i0, sem_i1):
    ew = EP // NW
    nch = ew // CE
    wid = lax.axis_index("s") * 2 + lax.axis_index("c")
    sem_s = (sem_s0, sem_s1)
    sem_r = (sem_r0, sem_r1)
    sem_o = (sem_o0, sem_o1)
    sem_i = (sem_i0, sem_i1)

    def fire_gathers(ch):
        p = ch % 2
        r0 = wid * (ew // 8) + ch * CP
        blk = r0 // BEF
        rloc = r0 % BEF
        cps = []
        for k in range(NG):
            start = blk * (BEF * 8) + k * BEF + rloc
            start = jnp.where(start >= E_EDGES, 0, start)
            start = pl.multiple_of(start, 8)
            cps.append(pltpu.async_copy(
                sids_hbm.at[pl.ds(start, G)], sidx_v.at[p, k], sem_i[p]))
            cps.append(pltpu.async_copy(
                rids_hbm.at[pl.ds(start, G)], ridx_v.at[p, k], sem_i[p]))
        for cp in cps:
            cp.wait()
        cps = []
        for j in range(NG):
            cps.append(pltpu.async_copy(
                ps_hbm.at[sidx_v.at[p, j]],
                gs_v.at[p, pl.ds(j * G, G)], sem_s[p]))
            cps.append(pltpu.async_copy(
                pr_hbm.at[ridx_v.at[p, j]],
                gr_v.at[p, pl.ds(j * G, G)], sem_r[p]))
        return cps

    gath = fire_gathers(0)
    outw = {}
    for ch in range(nch):
        p = ch % 2
        nxt = fire_gathers(ch + 1) if ch + 1 < nch else []
        for cp in gath:
            cp.wait()
        gath = nxt
        if ch >= 2:
            outw[ch - 2].wait()

        @plsc.parallel_loop(0, CP, 1, unroll=2)
        def _row(g):
            for k in range(8):
                e = k * G + g
                acc_v[p, g, pl.ds(16 * k, 16)] = (
                    gs_v[p, e, :] + gr_v[p, e, :])

        prow0 = pl.multiple_of((wid * ew + ch * CE) // 8, 8)
        outw[ch] = pltpu.async_copy(
            acc_v.at[p], out_hbm.at[pl.ds(prow0, CP)], sem_o[p])
    outw[nch - 2].wait()
    outw[nch - 1].wait()


def _final_body(edt_ref, wet_ref, gsum_ref, out_ref):
    eproj = jnp.dot(wet_ref[...], edt_ref[...],
                    preferred_element_type=jnp.float32)
    gpt = gsum_ref[...].T
    gt = jnp.concatenate(
        [gpt[16 * k:16 * (k + 1), :] for k in range(8)], axis=1)
    out_ref[...] = jnp.maximum(eproj + gt, 0.0)


def kernel(vdata, edata, sender_ids, receiver_ids, W, b):
    Bn, N, DV = vdata.shape
    _, E, DE = edata.shape
    DOUT = W.shape[1]

    vd = vdata.reshape(N, DV)
    edt = edata.reshape(E, DE).T
    be = BEF * 8
    sid = sender_ids.reshape(E)
    rid = receiver_ids.reshape(E)
    we = W[:DE]
    ws = W[DE:DE + DV]
    wr = W[DE + DV:]
    wet = we.T
    brow = b.reshape(1, DOUT)

    ps, pr = pl.pallas_call(
        _tables_body,
        out_shape=[jax.ShapeDtypeStruct((N, DOUT), jnp.float32)] * 2,
    )(vd, ws, wr, brow)

    mesh = plsc.VectorSubcoreMesh(core_axis_name="c", subcore_axis_name="s")
    sc_call = pl.kernel(
        _sc_body,
        out_type=jax.ShapeDtypeStruct((EP // 8, 128), jnp.float32),
        mesh=mesh,
        compiler_params=pltpu.CompilerParams(use_tc_tiling_on_sc=False),
        scratch_types=[
            pltpu.VMEM((2, NG, G), jnp.int32),
            pltpu.VMEM((2, NG, G), jnp.int32),
            pltpu.VMEM((2, CE, 16), jnp.float32),
            pltpu.VMEM((2, CE, 16), jnp.float32),
            pltpu.VMEM((2, CP, 128), jnp.float32),
            pltpu.SemaphoreType.DMA,
            pltpu.SemaphoreType.DMA,
            pltpu.SemaphoreType.DMA,
            pltpu.SemaphoreType.DMA,
            pltpu.SemaphoreType.DMA,
            pltpu.SemaphoreType.DMA,
            pltpu.SemaphoreType.DMA,
            pltpu.SemaphoreType.DMA,
        ],
    )
    gsum = sc_call(ps, pr, sid, rid)

    outt = pl.pallas_call(
        _final_body,
        grid=(pl.cdiv(E, be),),
        in_specs=[
            pl.BlockSpec((DE, be), lambda i: (0, i)),
            pl.BlockSpec((DOUT, DE), lambda i: (0, 0)),
            pl.BlockSpec((BEF, 128), lambda i: (i, 0)),
        ],
        out_specs=pl.BlockSpec((DOUT, be), lambda i: (0, i)),
        out_shape=jax.ShapeDtypeStruct((DOUT, E), jnp.float32),
    )(edt, wet, gsum)
    return outt.T.reshape(Bn, E, DOUT)
